# baseline (speedup 1.0000x reference)
import jax
import jax.numpy as jnp
from jax.experimental import pallas as pl
from jax.experimental.pallas import tpu as pltpu


def _body(wt0_ref, wt1_ref, sent_ref, mask_ref, out_ref):
    L = sent_ref.shape[1]
    wt0 = wt0_ref[...]
    wd = wt1_ref[...] - wt0
    mf = mask_ref[...].reshape(mask_ref.shape[0], -1).astype(jnp.float32)
    lane = jax.lax.broadcasted_iota(jnp.int32, (1, 128), 1)

    for g in range(L // 8):
        rolls = []
        membs = []
        for t in range(8):
            l = 8 * g + t
            s = sent_ref[:, l, :]
            rolls.append(jnp.roll(s, 16 * t, axis=1) if t else s)
            membs.append(wt0 + mf[:, l:l + 1] * wd)
        base = 1152 * g
        out_ref[:, base:base + 128] = rolls[0]
        for j in range(1, 8):
            col = jnp.where(lane < 16 * j, membs[j - 1], rolls[j])
            if j > 1:
                col = jnp.where(lane < 16 * (j - 1), rolls[j - 1], col)
            out_ref[:, base + 128 * j:base + 128 * (j + 1)] = col
        out_ref[:, base + 1024:base + 1152] = jnp.where(lane < 112, rolls[7], membs[7])


def kernel(sent, mask, mask_embed_weight):
    B, L, D = sent.shape
    M = mask_embed_weight.shape[1]
    F = L * (D + M)
    mask_i = mask.astype(jnp.int32)
    wt0 = jnp.tile(mask_embed_weight[0], 8).reshape(1, 8 * M)
    wt1 = jnp.tile(mask_embed_weight[1], 8).reshape(1, 8 * M)
    Bb = 64 if B % 64 == 0 else 8
    NJ = 5 if L % 40 == 0 else 1
    Lc = L // NJ
    mask4 = mask_i.reshape(B, NJ, 1, Lc)
    grid = (B // Bb, NJ)
    out = pl.pallas_call(
        _body,
        grid=grid,
        in_specs=[
            pl.BlockSpec((1, 8 * M), lambda i, j: (0, 0)),
            pl.BlockSpec((1, 8 * M), lambda i, j: (0, 0)),
            pl.BlockSpec((Bb, Lc, D), lambda i, j: (i, j, 0)),
            pl.BlockSpec((Bb, 1, 1, Lc), lambda i, j: (i, j, 0, 0)),
        ],
        out_specs=pl.BlockSpec((Bb, Lc * (D + M)), lambda i, j: (i, j)),
        out_shape=jax.ShapeDtypeStruct((B, F), jnp.float32),
        compiler_params=pltpu.CompilerParams(
            dimension_semantics=("parallel", "arbitrary"),
        ),
    )(wt0, wt1, sent, mask4)
    return out.reshape(B, L, D + M)


# speedup vs baseline: 2.0121x; 2.0121x over previous
import jax
import jax.numpy as jnp
from jax.experimental import pallas as pl
from jax.experimental.pallas import tpu as pltpu


def _body(wt0_ref, wt1_ref, sent_ref, mask_ref, out_ref):
    L = sent_ref.shape[1]
    wt0 = wt0_ref[...]
    wd = wt1_ref[...] - wt0
    mf = mask_ref[...].astype(jnp.float32)
    lane = jax.lax.broadcasted_iota(jnp.int32, (1, 128), 1)

    for g in range(L // 8):
        rolls = []
        membs = []
        for t in range(8):
            l = 8 * g + t
            s = sent_ref[:, l, :]
            rolls.append(jnp.roll(s, 16 * t, axis=1) if t else s)
            membs.append(wt0 + mf[:, l:l + 1] * wd)
        base = 1152 * g
        out_ref[:, base:base + 128] = rolls[0]
        for j in range(1, 8):
            col = jnp.where(lane < 16 * j, membs[j - 1], rolls[j])
            if j > 1:
                col = jnp.where(lane < 16 * (j - 1), rolls[j - 1], col)
            out_ref[:, base + 128 * j:base + 128 * (j + 1)] = col
        out_ref[:, base + 1024:base + 1152] = jnp.where(lane < 112, rolls[7], membs[7])


def kernel(sent, mask, mask_embed_weight):
    B, L, D = sent.shape
    M = mask_embed_weight.shape[1]
    F = L * (D + M)
    mask_i = mask.astype(jnp.int32)
    wt0 = jnp.tile(mask_embed_weight[0], 8).reshape(1, 8 * M)
    wt1 = jnp.tile(mask_embed_weight[1], 8).reshape(1, 8 * M)
    Bb = 64 if B % 64 == 0 else 8
    grid = (B // Bb,)
    out = pl.pallas_call(
        _body,
        grid=grid,
        in_specs=[
            pl.BlockSpec((1, 8 * M), lambda i: (0, 0)),
            pl.BlockSpec((1, 8 * M), lambda i: (0, 0)),
            pl.BlockSpec((Bb, L, D), lambda i: (i, 0, 0)),
            pl.BlockSpec((Bb, L), lambda i: (i, 0)),
        ],
        out_specs=pl.BlockSpec((Bb, F), lambda i: (i, 0)),
        out_shape=jax.ShapeDtypeStruct((B, F), jnp.float32),
        compiler_params=pltpu.CompilerParams(
            dimension_semantics=("parallel",),
        ),
    )(wt0, wt1, sent, mask_i)
    return out.reshape(B, L, D + M)
